# baseline (speedup 1.0000x reference)
import functools

import jax
import jax.numpy as jnp
from jax import lax
from jax.experimental import pallas as pl
from jax.experimental.pallas import tpu as pltpu
from jax.experimental.pallas import tpu_sc as plsc

N = 10000
E = 320000
D_IN = 128
D_H = 64
W = 128

NC = 2
NS = 16
NW = NC * NS
CHUNK = 128
J = 80
EPW = J * CHUNK
E_PAD = EPW * NW
N_ACC = 10112
RPS = N_ACC // NS

_mesh = plsc.VectorSubcoreMesh(core_axis_name="c", subcore_axis_name="s")


@functools.partial(
    pl.kernel,
    out_type=jax.ShapeDtypeStruct((NC, N_ACC, W), jnp.float32),
    mesh=_mesh,
    scratch_types=[
        pltpu.VMEM((J, CHUNK), jnp.int32),
        pltpu.VMEM((CHUNK, W), jnp.float32),
        pltpu.VMEM_SHARED((N_ACC, W), jnp.float32),
    ],
)
def _sc_degree(colp_hbm, zerosw_hbm, onesw_hbm, out_hbm, colv, onesv, acc):
    c = lax.axis_index("c")
    s = lax.axis_index("s")
    wid = c * NS + s
    pltpu.sync_copy(zerosw_hbm.at[pl.ds(s * RPS, RPS)], acc.at[pl.ds(s * RPS, RPS)])
    pltpu.sync_copy(onesw_hbm, onesv)
    pltpu.sync_copy(colp_hbm.at[wid], colv)
    plsc.subcore_barrier()

    def body(j, carry):
        pltpu.sync_copy(onesv, acc.at[colv.at[j]], add=True)
        return carry

    lax.fori_loop(0, J, body, 0)
    plsc.subcore_barrier()
    pltpu.sync_copy(acc.at[pl.ds(s * RPS, RPS)], out_hbm.at[c, pl.ds(s * RPS, RPS)])


@functools.partial(
    pl.kernel,
    out_type=jax.ShapeDtypeStruct((NC, N_ACC, W), jnp.float32),
    mesh=_mesh,
    scratch_types=[
        pltpu.VMEM((J, CHUNK), jnp.int32),
        pltpu.VMEM((J, CHUNK), jnp.int32),
        pltpu.VMEM((CHUNK, W), jnp.float32),
        pltpu.VMEM_SHARED((N_ACC, W), jnp.float32),
        pltpu.SemaphoreType.DMA,
    ],
)
def _sc_scatter(g_hbm, rowp_hbm, colp_hbm, zerosw_hbm, out_hbm,
                rowv, colv, buf, acc, sem):
    c = lax.axis_index("c")
    s = lax.axis_index("s")
    wid = c * NS + s
    pltpu.sync_copy(zerosw_hbm.at[pl.ds(s * RPS, RPS)], acc.at[pl.ds(s * RPS, RPS)])
    pltpu.sync_copy(rowp_hbm.at[wid], rowv)
    pltpu.sync_copy(colp_hbm.at[wid], colv)
    plsc.subcore_barrier()

    def body(j, carry):
        pltpu.async_copy(g_hbm.at[rowv.at[j]], buf, sem).wait()
        pltpu.sync_copy(buf, acc.at[colv.at[j]], add=True)
        return carry

    lax.fori_loop(0, J, body, 0)
    plsc.subcore_barrier()
    pltpu.sync_copy(acc.at[pl.ds(s * RPS, RPS)], out_hbm.at[c, pl.ds(s * RPS, RPS)])


_RB = 632


def _tc_prep_body(deg_ref, x_ref, w1_ref, dinv_ref, g1_ref):
    dp = deg_ref[...]
    d = dp[0, :, :1] + dp[1, :, :1] + 1.0
    dinv = lax.rsqrt(d)
    dinv_b = jnp.broadcast_to(dinv, (_RB, D_H))
    h1 = lax.dot_general(x_ref[...], w1_ref[...],
                         (((1,), (1,)), ((), ())),
                         preferred_element_type=jnp.float32)
    dinv_ref[...] = dinv_b
    g1_ref[...] = jnp.concatenate(
        [dinv_b * h1, jnp.zeros((_RB, W - D_H), jnp.float32)], axis=1)


_tc_prep = pl.pallas_call(
    _tc_prep_body,
    grid=(N_ACC // _RB,),
    in_specs=[
        pl.BlockSpec((NC, _RB, W), lambda i: (0, i, 0)),
        pl.BlockSpec((_RB, D_IN), lambda i: (i, 0)),
        pl.BlockSpec((D_H, D_IN), lambda i: (0, 0)),
    ],
    out_specs=[
        pl.BlockSpec((_RB, D_H), lambda i: (i, 0)),
        pl.BlockSpec((_RB, W), lambda i: (i, 0)),
    ],
    out_shape=[
        jax.ShapeDtypeStruct((N_ACC, D_H), jnp.float32),
        jax.ShapeDtypeStruct((N_ACC, W), jnp.float32),
    ],
)


def _leaky(h):
    return jnp.where(h >= 0, h, 0.01 * h)


def _tc_mid_body(p_ref, gprev_ref, dinv_ref, b_ref, w_ref, gnext_ref):
    pp = p_ref[...][:, :, :D_H]
    dinv_b = dinv_ref[...]
    t = _leaky(dinv_b * (pp[0] + pp[1] + gprev_ref[...][:, :D_H]) + b_ref[...])
    h = lax.dot_general(t, w_ref[...], (((1,), (1,)), ((), ())),
                        preferred_element_type=jnp.float32)
    gnext_ref[...] = jnp.concatenate(
        [dinv_b * h, jnp.zeros((_RB, W - D_H), jnp.float32)], axis=1)


_tc_mid = pl.pallas_call(
    _tc_mid_body,
    grid=(N_ACC // _RB,),
    in_specs=[
        pl.BlockSpec((NC, _RB, W), lambda i: (0, i, 0)),
        pl.BlockSpec((_RB, W), lambda i: (i, 0)),
        pl.BlockSpec((_RB, D_H), lambda i: (i, 0)),
        pl.BlockSpec((1, D_H), lambda i: (0, 0)),
        pl.BlockSpec((D_H, D_H), lambda i: (0, 0)),
    ],
    out_specs=pl.BlockSpec((_RB, W), lambda i: (i, 0)),
    out_shape=jax.ShapeDtypeStruct((N_ACC, W), jnp.float32),
)


_RBF = 1000


def _tc_final_body(p_ref, gprev_ref, dinv_ref, b_ref, out_ref):
    pp = p_ref[...][:, :, :D_H]
    out_ref[...] = _leaky(
        dinv_ref[...] * (pp[0] + pp[1] + gprev_ref[...][:, :D_H]) + b_ref[...])


_tc_final = pl.pallas_call(
    _tc_final_body,
    grid=(N // _RBF,),
    in_specs=[
        pl.BlockSpec((NC, _RBF, W), lambda i: (0, i, 0)),
        pl.BlockSpec((_RBF, W), lambda i: (i, 0)),
        pl.BlockSpec((_RBF, D_H), lambda i: (i, 0)),
        pl.BlockSpec((1, D_H), lambda i: (0, 0)),
    ],
    out_specs=pl.BlockSpec((_RBF, D_H), lambda i: (i, 0)),
    out_shape=jax.ShapeDtypeStruct((N, D_H), jnp.float32),
)


def kernel(x, edge_index, batch, W1, b1, W2, b2, W3, b3):
    row = edge_index[0]
    col = edge_index[1]
    pad = E_PAD - E
    pad_row = (jnp.arange(pad, dtype=jnp.int32) * 8) % N
    pad_col = N + (jnp.arange(pad, dtype=jnp.int32) % (N_ACC - N))
    rowp = jnp.concatenate([row, pad_row]).reshape(NW, J, CHUNK)
    colp = jnp.concatenate([col, pad_col]).reshape(NW, J, CHUNK)

    xp = jnp.pad(x, ((0, N_ACC - N), (0, 0)))
    zerosw = jnp.zeros((N_ACC, W), jnp.float32)
    onesw = jnp.ones((CHUNK, W), jnp.float32)

    degp = _sc_degree(colp, zerosw, onesw)
    dinv_b, g1 = _tc_prep(degp, xp, W1)
    p1 = _sc_scatter(g1, rowp, colp, zerosw)
    g2 = _tc_mid(p1, g1, dinv_b, b1.reshape(1, D_H), W2)
    p2 = _sc_scatter(g2, rowp, colp, zerosw)
    g3 = _tc_mid(p2, g2, dinv_b, b2.reshape(1, D_H), W3)
    p3 = _sc_scatter(g3, rowp, colp, zerosw)
    return _tc_final(p3, g3, dinv_b, b3.reshape(1, D_H))


# speedup vs baseline: 1.0088x; 1.0088x over previous
import functools

import jax
import jax.numpy as jnp
from jax import lax
from jax.experimental import pallas as pl
from jax.experimental.pallas import tpu as pltpu
from jax.experimental.pallas import tpu_sc as plsc

N = 10000
E = 320000
D_IN = 128
D_H = 64
W = 128

NC = 2
NS = 16
NW = NC * NS
CHUNK = 128
J = 79
EPW = J * CHUNK
E_PAD = EPW * NW
N_ACC = 10112
RPS = N_ACC // NS

_mesh = plsc.VectorSubcoreMesh(core_axis_name="c", subcore_axis_name="s")


@functools.partial(
    pl.kernel,
    out_type=jax.ShapeDtypeStruct((NC, N_ACC, W), jnp.float32),
    mesh=_mesh,
    scratch_types=[
        pltpu.VMEM((J, CHUNK), jnp.int32),
        pltpu.VMEM((CHUNK, W), jnp.float32),
        pltpu.VMEM_SHARED((N_ACC, W), jnp.float32),
    ],
)
def _sc_degree(colp_hbm, zerosw_hbm, onesw_hbm, out_hbm, colv, onesv, acc):
    c = lax.axis_index("c")
    s = lax.axis_index("s")
    wid = c * NS + s
    pltpu.sync_copy(zerosw_hbm.at[pl.ds(s * RPS, RPS)], acc.at[pl.ds(s * RPS, RPS)])
    pltpu.sync_copy(onesw_hbm, onesv)
    pltpu.sync_copy(colp_hbm.at[wid], colv)
    plsc.subcore_barrier()

    def body(j, carry):
        pltpu.sync_copy(onesv, acc.at[colv.at[j]], add=True)
        return carry

    lax.fori_loop(0, J, body, 0)
    plsc.subcore_barrier()
    pltpu.sync_copy(acc.at[pl.ds(s * RPS, RPS)], out_hbm.at[c, pl.ds(s * RPS, RPS)])


@functools.partial(
    pl.kernel,
    out_type=jax.ShapeDtypeStruct((NC, N_ACC, W), jnp.float32),
    mesh=_mesh,
    scratch_types=[
        pltpu.VMEM((J, CHUNK), jnp.int32),
        pltpu.VMEM((J, CHUNK), jnp.int32),
        pltpu.VMEM((CHUNK, W), jnp.float32),
        pltpu.VMEM_SHARED((N_ACC, W), jnp.float32),
        pltpu.SemaphoreType.DMA,
    ],
)
def _sc_scatter(g_hbm, rowp_hbm, colp_hbm, zerosw_hbm, out_hbm,
                rowv, colv, buf, acc, sem):
    c = lax.axis_index("c")
    s = lax.axis_index("s")
    wid = c * NS + s
    pltpu.sync_copy(zerosw_hbm.at[pl.ds(s * RPS, RPS)], acc.at[pl.ds(s * RPS, RPS)])
    pltpu.sync_copy(rowp_hbm.at[wid], rowv)
    pltpu.sync_copy(colp_hbm.at[wid], colv)
    plsc.subcore_barrier()

    def body(j, carry):
        pltpu.async_copy(g_hbm.at[rowv.at[j]], buf, sem).wait()
        pltpu.sync_copy(buf, acc.at[colv.at[j]], add=True)
        return carry

    lax.fori_loop(0, J, body, 0)
    plsc.subcore_barrier()
    pltpu.sync_copy(acc.at[pl.ds(s * RPS, RPS)], out_hbm.at[c, pl.ds(s * RPS, RPS)])


_RB = 632


def _tc_prep_body(deg_ref, x_ref, w1_ref, dinv_ref, g1_ref):
    dp = deg_ref[...]
    d = dp[0, :, :1] + dp[1, :, :1] + 1.0
    dinv = lax.rsqrt(d)
    dinv_b = jnp.broadcast_to(dinv, (_RB, D_H))
    h1 = lax.dot_general(x_ref[...], w1_ref[...],
                         (((1,), (1,)), ((), ())),
                         preferred_element_type=jnp.float32)
    dinv_ref[...] = dinv_b
    g1_ref[...] = jnp.concatenate(
        [dinv_b * h1, jnp.zeros((_RB, W - D_H), jnp.float32)], axis=1)


_tc_prep = pl.pallas_call(
    _tc_prep_body,
    grid=(N_ACC // _RB,),
    in_specs=[
        pl.BlockSpec((NC, _RB, W), lambda i: (0, i, 0)),
        pl.BlockSpec((_RB, D_IN), lambda i: (i, 0)),
        pl.BlockSpec((D_H, D_IN), lambda i: (0, 0)),
    ],
    out_specs=[
        pl.BlockSpec((_RB, D_H), lambda i: (i, 0)),
        pl.BlockSpec((_RB, W), lambda i: (i, 0)),
    ],
    out_shape=[
        jax.ShapeDtypeStruct((N_ACC, D_H), jnp.float32),
        jax.ShapeDtypeStruct((N_ACC, W), jnp.float32),
    ],
)


def _leaky(h):
    return jnp.where(h >= 0, h, 0.01 * h)


def _tc_mid_body(p_ref, gprev_ref, dinv_ref, b_ref, w_ref, gnext_ref):
    pp = p_ref[...][:, :, :D_H]
    dinv_b = dinv_ref[...]
    t = _leaky(dinv_b * (pp[0] + pp[1] + gprev_ref[...][:, :D_H]) + b_ref[...])
    h = lax.dot_general(t, w_ref[...], (((1,), (1,)), ((), ())),
                        preferred_element_type=jnp.float32)
    gnext_ref[...] = jnp.concatenate(
        [dinv_b * h, jnp.zeros((_RB, W - D_H), jnp.float32)], axis=1)


_tc_mid = pl.pallas_call(
    _tc_mid_body,
    grid=(N_ACC // _RB,),
    in_specs=[
        pl.BlockSpec((NC, _RB, W), lambda i: (0, i, 0)),
        pl.BlockSpec((_RB, W), lambda i: (i, 0)),
        pl.BlockSpec((_RB, D_H), lambda i: (i, 0)),
        pl.BlockSpec((1, D_H), lambda i: (0, 0)),
        pl.BlockSpec((D_H, D_H), lambda i: (0, 0)),
    ],
    out_specs=pl.BlockSpec((_RB, W), lambda i: (i, 0)),
    out_shape=jax.ShapeDtypeStruct((N_ACC, W), jnp.float32),
)


_RBF = 1000


def _tc_final_body(p_ref, gprev_ref, dinv_ref, b_ref, out_ref):
    pp = p_ref[...][:, :, :D_H]
    out_ref[...] = _leaky(
        dinv_ref[...] * (pp[0] + pp[1] + gprev_ref[...][:, :D_H]) + b_ref[...])


_tc_final = pl.pallas_call(
    _tc_final_body,
    grid=(N // _RBF,),
    in_specs=[
        pl.BlockSpec((NC, _RBF, W), lambda i: (0, i, 0)),
        pl.BlockSpec((_RBF, W), lambda i: (i, 0)),
        pl.BlockSpec((_RBF, D_H), lambda i: (i, 0)),
        pl.BlockSpec((1, D_H), lambda i: (0, 0)),
    ],
    out_specs=pl.BlockSpec((_RBF, D_H), lambda i: (i, 0)),
    out_shape=jax.ShapeDtypeStruct((N, D_H), jnp.float32),
)


def kernel(x, edge_index, batch, W1, b1, W2, b2, W3, b3):
    row = edge_index[0]
    col = edge_index[1]
    pad = E_PAD - E
    pad_row = (jnp.arange(pad, dtype=jnp.int32) * 8) % N
    pad_col = N + (jnp.arange(pad, dtype=jnp.int32) % (N_ACC - N))
    rowp = jnp.concatenate([row, pad_row]).reshape(NW, J, CHUNK)
    colp = jnp.concatenate([col, pad_col]).reshape(NW, J, CHUNK)

    xp = jnp.pad(x, ((0, N_ACC - N), (0, 0)))
    zerosw = jnp.zeros((N_ACC, W), jnp.float32)
    onesw = jnp.ones((CHUNK, W), jnp.float32)

    degp = _sc_degree(colp, zerosw, onesw)
    dinv_b, g1 = _tc_prep(degp, xp, W1)
    p1 = _sc_scatter(g1, rowp, colp, zerosw)
    g2 = _tc_mid(p1, g1, dinv_b, b1.reshape(1, D_H), W2)
    p2 = _sc_scatter(g2, rowp, colp, zerosw)
    g3 = _tc_mid(p2, g2, dinv_b, b2.reshape(1, D_H), W3)
    p3 = _sc_scatter(g3, rowp, colp, zerosw)
    return _tc_final(p3, g3, dinv_b, b3.reshape(1, D_H))
